# baseline (speedup 1.0000x reference)
import jax
import jax.numpy as jnp
from jax import lax
from jax.experimental import pallas as pl
from jax.experimental.pallas import tpu as pltpu
from jax.experimental.pallas import tpu_sc as plsc

NUM_CORES = 2
NUM_SUBCORES = 16
NUM_WORKERS = NUM_CORES * NUM_SUBCORES

BLK = 128

NB = 4


def _make_mesh():
    return plsc.VectorSubcoreMesh(
        core_axis_name="c", subcore_axis_name="s",
        num_cores=NUM_CORES, num_subcores=NUM_SUBCORES)


def _transpose_table(tT, tail, V, D):
    n_blocks = V // BLK
    tail_rows = V - n_blocks * BLK
    tail_base = n_blocks * BLK
    n_iter = -(-n_blocks // NUM_WORKERS)
    n_iter += n_iter % 2

    @pl.kernel(
        mesh=_make_mesh(),
        compiler_params=pltpu.CompilerParams(
            use_tc_tiling_on_sc=True, needs_layout_passes=False),
        out_type=jax.ShapeDtypeStruct((V, 2 * D), jnp.float32),
        scratch_types=[
            pltpu.VMEM((D, BLK), jnp.float32),
            pltpu.VMEM((D, BLK), jnp.float32),
            pltpu.VMEM((BLK, 2 * D), jnp.float32),
            pltpu.VMEM((BLK, 2 * D), jnp.float32),
            pltpu.VMEM((tail_rows, D), jnp.float32),
            pltpu.SemaphoreType.DMA,
            pltpu.SemaphoreType.DMA,
        ],
    )
    def k0(tT_hbm, tail_hbm, fmt_hbm, s0, s1, w0, w1, tv, sin, sout):
        wid = lax.axis_index("s") * NUM_CORES + lax.axis_index("c")
        lane16 = jnp.arange(16, dtype=jnp.int32)

        @pl.when(wid == NUM_WORKERS - 1)
        def _():
            pltpu.sync_copy(tail_hbm, tv)

            def trow(r, carry):
                for j in range(D // 16):
                    w0[r, pl.ds(16 * j, 16)] = tv[r, pl.ds(16 * j, 16)]
                return carry
            lax.fori_loop(0, tail_rows, trow, 0)
            pltpu.sync_copy(w0.at[pl.ds(0, tail_rows)],
                            fmt_hbm.at[pl.ds(tail_base, tail_rows)])

        def cid(j):
            return j * NUM_WORKERS + wid

        def fire_in(c, stag):
            @pl.when(c < n_blocks)
            def _():
                pltpu.async_copy(tT_hbm.at[:, pl.ds(c * BLK, BLK)], stag, sin)

        def wait_in(c, stag):
            @pl.when(c < n_blocks)
            def _():
                pltpu.make_async_copy(
                    tT_hbm.at[:, pl.ds(c * BLK, BLK)], stag, sin).wait()

        def transpose(c, stag, wbuf):
            @pl.when(c < n_blocks)
            def _():
                def row(i, carry):
                    col = jnp.full((16,), i, jnp.int32)
                    for j in range(D // 16):
                        v = plsc.load_gather(stag, [lane16 + 16 * j, col])
                        wbuf[i, pl.ds(16 * j, 16)] = v
                    return carry
                lax.fori_loop(0, BLK, row, 0)

        def fire_out(c, wbuf):
            @pl.when(c < n_blocks)
            def _():
                pltpu.async_copy(
                    wbuf, fmt_hbm.at[pl.ds(c * BLK, BLK)], sout)

        def wait_out(c, wbuf):
            @pl.when(jnp.logical_and(c >= 0, c < n_blocks))
            def _():
                pltpu.make_async_copy(
                    wbuf, fmt_hbm.at[pl.ds(c * BLK, BLK)], sout).wait()

        fire_in(cid(0), s0)

        def body(i, carry):
            ca = cid(2 * i)
            cb = cid(2 * i + 1)
            fire_in(cb, s1)
            wait_in(ca, s0)
            wait_out(ca - 2 * NUM_WORKERS, w0)
            transpose(ca, s0, w0)
            fire_out(ca, w0)
            fire_in(cid(2 * i + 2), s0)
            wait_in(cb, s1)
            wait_out(cb - 2 * NUM_WORKERS, w1)
            transpose(cb, s1, w1)
            fire_out(cb, w1)
            return carry

        lax.fori_loop(0, n_iter // 2, body, 0)
        wait_out(cid(n_iter - 2), w0)
        wait_out(cid(n_iter - 1), w1)

    return k0(tT, tail)


def _gather(fmt, idx, BATCH, HIST, D):
    batches_per_w = BATCH // NUM_WORKERS
    n_stages = batches_per_w // NB
    assert BATCH % NUM_WORKERS == 0 and batches_per_w % NB == 0
    assert n_stages % 2 == 0

    @pl.kernel(
        mesh=_make_mesh(),
        compiler_params=pltpu.CompilerParams(use_tc_tiling_on_sc=True),
        out_type=jax.ShapeDtypeStruct((BATCH, HIST, D), jnp.float32),
        scratch_types=[
            pltpu.VMEM((batches_per_w, HIST), jnp.int32),
            pltpu.VMEM((NB, HIST, 2 * D), jnp.float32),
            pltpu.VMEM((NB, HIST, 2 * D), jnp.float32),
            pltpu.VMEM((NB, HIST, D), jnp.float32),
            pltpu.SemaphoreType.DMA,
            pltpu.SemaphoreType.DMA,
        ],
    )
    def k2(fmt_hbm, idx_hbm, out_hbm, idx_v, buf0, buf1, obuf, sem0, sem1):
        wid = lax.axis_index("s") * NUM_CORES + lax.axis_index("c")
        base = wid * batches_per_w
        pltpu.sync_copy(idx_hbm.at[pl.ds(base, batches_per_w)], idx_v)

        def fire(st, buf, sem):
            for b in range(NB):
                pltpu.async_copy(
                    fmt_hbm.at[idx_v.at[st * NB + b]], buf.at[b], sem)

        def drain_out(st, buf, sem):
            for b in range(NB):
                pltpu.make_async_copy(
                    fmt_hbm.at[idx_v.at[st * NB + b]], buf.at[b], sem).wait()
            for b in range(NB):
                def row(h, carry):
                    for j in range(D // 16):
                        obuf[b, h, pl.ds(16 * j, 16)] = \
                            buf[b, h, pl.ds(16 * j, 16)]
                    return carry
                lax.fori_loop(0, HIST, row, 0)
            pltpu.sync_copy(obuf, out_hbm.at[pl.ds(base + st * NB, NB)])

        fire(0, buf0, sem0)

        def body(i, carry):
            s0 = 2 * i
            fire(s0 + 1, buf1, sem1)
            drain_out(s0, buf0, sem0)

            @pl.when(s0 + 2 < n_stages)
            def _():
                fire(s0 + 2, buf0, sem0)

            drain_out(s0 + 1, buf1, sem1)
            return carry

        lax.fori_loop(0, n_stages // 2, body, 0)

    return k2(fmt, idx)


def kernel(inputs, embedding):
    batch, hist = inputs.shape
    num_emb, feat = embedding.shape
    emb = jnp.asarray(embedding, jnp.float32)
    tail_base = (num_emb // BLK) * BLK
    fmt = _transpose_table(emb.T, emb[tail_base:], num_emb, feat)
    return _gather(fmt, inputs.astype(jnp.int32), batch, hist, feat)


# speedup vs baseline: 1.2061x; 1.2061x over previous
import jax
import jax.numpy as jnp
from jax import lax
from jax.experimental import pallas as pl
from jax.experimental.pallas import tpu as pltpu
from jax.experimental.pallas import tpu_sc as plsc

NUM_CORES = 2
NUM_SUBCORES = 16
NUM_WORKERS = NUM_CORES * NUM_SUBCORES

BLK = 128

NB = 4


def _make_mesh():
    return plsc.VectorSubcoreMesh(
        core_axis_name="c", subcore_axis_name="s",
        num_cores=NUM_CORES, num_subcores=NUM_SUBCORES)


def _transpose_table(tT, tail, V, D):
    n_blocks = V // BLK
    tail_rows = V - n_blocks * BLK
    tail_base = n_blocks * BLK
    n_iter = -(-n_blocks // NUM_WORKERS)
    n_iter += n_iter % 2

    @pl.kernel(
        mesh=_make_mesh(),
        compiler_params=pltpu.CompilerParams(
            use_tc_tiling_on_sc=True, needs_layout_passes=False),
        out_type=jax.ShapeDtypeStruct((V, 2 * D), jnp.float32),
        scratch_types=[
            pltpu.VMEM((D, BLK), jnp.float32),
            pltpu.VMEM((D, BLK), jnp.float32),
            pltpu.VMEM((BLK, 2 * D), jnp.float32),
            pltpu.VMEM((BLK, 2 * D), jnp.float32),
            pltpu.VMEM((tail_rows, D), jnp.float32),
            pltpu.SemaphoreType.DMA,
            pltpu.SemaphoreType.DMA,
        ],
    )
    def k0(tT_hbm, tail_hbm, fmt_hbm, s0, s1, w0, w1, tv, sin, sout):
        wid = lax.axis_index("s") * NUM_CORES + lax.axis_index("c")
        lane16 = jnp.arange(16, dtype=jnp.int32)

        @pl.when(wid == NUM_WORKERS - 1)
        def _():
            pltpu.sync_copy(tail_hbm, tv)

            def trow(r, carry):
                for j in range(D // 16):
                    w0[r, pl.ds(16 * j, 16)] = tv[r, pl.ds(16 * j, 16)]
                return carry
            lax.fori_loop(0, tail_rows, trow, 0)
            pltpu.sync_copy(w0.at[pl.ds(0, tail_rows)],
                            fmt_hbm.at[pl.ds(tail_base, tail_rows)])

        def cid(j):
            return j * NUM_WORKERS + wid

        def fire_in(c, stag):
            @pl.when(c < n_blocks)
            def _():
                pltpu.async_copy(tT_hbm.at[:, pl.ds(c * BLK, BLK)], stag, sin)

        def wait_in(c, stag):
            @pl.when(c < n_blocks)
            def _():
                pltpu.make_async_copy(
                    tT_hbm.at[:, pl.ds(c * BLK, BLK)], stag, sin).wait()

        def transpose(c, stag, wbuf):
            @pl.when(c < n_blocks)
            def _():
                def grp(m, carry):
                    rows16 = lane16 + 16 * m
                    for f in range(D):
                        v = stag[f, pl.ds(16 * m, 16)]
                        plsc.store_scatter(
                            wbuf, [rows16, jnp.full((16,), f, jnp.int32)], v)
                    return carry
                lax.fori_loop(0, BLK // 16, grp, 0)

        def fire_out(c, wbuf):
            @pl.when(c < n_blocks)
            def _():
                pltpu.async_copy(
                    wbuf, fmt_hbm.at[pl.ds(c * BLK, BLK)], sout)

        def wait_out(c, wbuf):
            @pl.when(jnp.logical_and(c >= 0, c < n_blocks))
            def _():
                pltpu.make_async_copy(
                    wbuf, fmt_hbm.at[pl.ds(c * BLK, BLK)], sout).wait()

        fire_in(cid(0), s0)

        def body(i, carry):
            ca = cid(2 * i)
            cb = cid(2 * i + 1)
            fire_in(cb, s1)
            wait_in(ca, s0)
            wait_out(ca - 2 * NUM_WORKERS, w0)
            transpose(ca, s0, w0)
            fire_out(ca, w0)
            fire_in(cid(2 * i + 2), s0)
            wait_in(cb, s1)
            wait_out(cb - 2 * NUM_WORKERS, w1)
            transpose(cb, s1, w1)
            fire_out(cb, w1)
            return carry

        lax.fori_loop(0, n_iter // 2, body, 0)
        wait_out(cid(n_iter - 2), w0)
        wait_out(cid(n_iter - 1), w1)

    return k0(tT, tail)


def _gather(fmt, idx, BATCH, HIST, D):
    batches_per_w = BATCH // NUM_WORKERS
    n_stages = batches_per_w // NB
    assert BATCH % NUM_WORKERS == 0 and batches_per_w % NB == 0
    assert n_stages % 2 == 0

    @pl.kernel(
        mesh=_make_mesh(),
        compiler_params=pltpu.CompilerParams(use_tc_tiling_on_sc=True),
        out_type=jax.ShapeDtypeStruct((BATCH, HIST, D), jnp.float32),
        scratch_types=[
            pltpu.VMEM((batches_per_w, HIST), jnp.int32),
            pltpu.VMEM((NB, HIST, 2 * D), jnp.float32),
            pltpu.VMEM((NB, HIST, 2 * D), jnp.float32),
            pltpu.VMEM((NB, HIST, D), jnp.float32),
            pltpu.SemaphoreType.DMA,
            pltpu.SemaphoreType.DMA,
        ],
    )
    def k2(fmt_hbm, idx_hbm, out_hbm, idx_v, buf0, buf1, obuf, sem0, sem1):
        wid = lax.axis_index("s") * NUM_CORES + lax.axis_index("c")
        base = wid * batches_per_w
        pltpu.sync_copy(idx_hbm.at[pl.ds(base, batches_per_w)], idx_v)

        def fire(st, buf, sem):
            for b in range(NB):
                pltpu.async_copy(
                    fmt_hbm.at[idx_v.at[st * NB + b]], buf.at[b], sem)

        def drain_out(st, buf, sem):
            for b in range(NB):
                pltpu.make_async_copy(
                    fmt_hbm.at[idx_v.at[st * NB + b]], buf.at[b], sem).wait()
            for b in range(NB):
                def row(h, carry):
                    for j in range(D // 16):
                        obuf[b, h, pl.ds(16 * j, 16)] = \
                            buf[b, h, pl.ds(16 * j, 16)]
                    return carry
                lax.fori_loop(0, HIST, row, 0)
            pltpu.sync_copy(obuf, out_hbm.at[pl.ds(base + st * NB, NB)])

        fire(0, buf0, sem0)

        def body(i, carry):
            s0 = 2 * i
            fire(s0 + 1, buf1, sem1)
            drain_out(s0, buf0, sem0)

            @pl.when(s0 + 2 < n_stages)
            def _():
                fire(s0 + 2, buf0, sem0)

            drain_out(s0 + 1, buf1, sem1)
            return carry

        lax.fori_loop(0, n_stages // 2, body, 0)

    return k2(fmt, idx)


def kernel(inputs, embedding):
    batch, hist = inputs.shape
    num_emb, feat = embedding.shape
    emb = jnp.asarray(embedding, jnp.float32)
    tail_base = (num_emb // BLK) * BLK
    fmt = _transpose_table(emb.T, emb[tail_base:], num_emb, feat)
    return _gather(fmt, inputs.astype(jnp.int32), batch, hist, feat)


# speedup vs baseline: 2.3811x; 1.9741x over previous
import jax
import jax.numpy as jnp
from jax import lax
from jax.experimental import pallas as pl
from jax.experimental.pallas import tpu as pltpu
from jax.experimental.pallas import tpu_sc as plsc

NUM_CORES = 2
NUM_SUBCORES = 16
NUM_WORKERS = NUM_CORES * NUM_SUBCORES

NB = 4


def _make_mesh():
    return plsc.VectorSubcoreMesh(
        core_axis_name="c", subcore_axis_name="s",
        num_cores=NUM_CORES, num_subcores=NUM_SUBCORES)


def _gather(fmt, idx, BATCH, HIST, D):
    batches_per_w = BATCH // NUM_WORKERS
    n_stages = batches_per_w // NB
    assert BATCH % NUM_WORKERS == 0 and batches_per_w % NB == 0
    assert n_stages % 2 == 0

    @pl.kernel(
        mesh=_make_mesh(),
        compiler_params=pltpu.CompilerParams(use_tc_tiling_on_sc=True),
        out_type=jax.ShapeDtypeStruct((BATCH, HIST, D), jnp.float32),
        scratch_types=[
            pltpu.VMEM((batches_per_w, HIST), jnp.int32),
            pltpu.VMEM((NB, HIST, 2 * D), jnp.float32),
            pltpu.VMEM((NB, HIST, 2 * D), jnp.float32),
            pltpu.VMEM((NB, HIST, D), jnp.float32),
            pltpu.SemaphoreType.DMA,
            pltpu.SemaphoreType.DMA,
        ],
    )
    def k2(fmt_hbm, idx_hbm, out_hbm, idx_v, buf0, buf1, obuf, sem0, sem1):
        wid = lax.axis_index("s") * NUM_CORES + lax.axis_index("c")
        base = wid * batches_per_w
        pltpu.sync_copy(idx_hbm.at[pl.ds(base, batches_per_w)], idx_v)

        def fire(st, buf, sem):
            for b in range(NB):
                pltpu.async_copy(
                    fmt_hbm.at[idx_v.at[st * NB + b]], buf.at[b], sem)

        def drain_out(st, buf, sem):
            for b in range(NB):
                pltpu.make_async_copy(
                    fmt_hbm.at[idx_v.at[st * NB + b]], buf.at[b], sem).wait()
            for b in range(NB):
                def row(h, carry):
                    for j in range(D // 16):
                        obuf[b, h, pl.ds(16 * j, 16)] = \
                            buf[b, h, pl.ds(16 * j, 16)]
                    return carry
                lax.fori_loop(0, HIST, row, 0)
            pltpu.sync_copy(obuf, out_hbm.at[pl.ds(base + st * NB, NB)])

        fire(0, buf0, sem0)

        def body(i, carry):
            s0 = 2 * i
            fire(s0 + 1, buf1, sem1)
            drain_out(s0, buf0, sem0)

            @pl.when(s0 + 2 < n_stages)
            def _():
                fire(s0 + 2, buf0, sem0)

            drain_out(s0 + 1, buf1, sem1)
            return carry

        lax.fori_loop(0, n_stages // 2, body, 0)

    return k2(fmt, idx)


def kernel(inputs, embedding):
    batch, hist = inputs.shape
    num_emb, feat = embedding.shape
    emb = jnp.asarray(embedding, jnp.float32)
    fmt = jnp.pad(emb, ((0, 0), (0, feat)))
    return _gather(fmt, inputs.astype(jnp.int32), batch, hist, feat)
